# baseline (speedup 1.0000x reference)
import functools

import jax
import jax.numpy as jnp
from jax import lax
from jax.experimental import pallas as pl
from jax.experimental.pallas import tpu as pltpu
from jax.experimental.pallas import tpu_sc as plsc

N = 10000
D = 256
HALF = 128
E = 160000
NC = 2
NS = 16
CH = 128
NPH = 2
CPP = 40
EPT = E // NS
PADT = NPH * CPP * CH - EPT
RPT = 624
RPT_LAST = N - 15 * RPT
BN = 1000
NBLK = N // BN

_mesh = plsc.VectorSubcoreMesh(core_axis_name="c", subcore_axis_name="s")


def _zero_acc(zeros_hbm, acc_sp, s):
    @pl.when(s < NS - 1)
    def _():
        pltpu.sync_copy(zeros_hbm.at[pl.ds(0, RPT)], acc_sp.at[pl.ds(s * RPT, RPT)])

    @pl.when(s == NS - 1)
    def _():
        pltpu.sync_copy(zeros_hbm, acc_sp.at[pl.ds(15 * RPT, RPT_LAST)])


def _writeback(acc_sp, acc_hbm, c, s):
    @pl.when(s < NS - 1)
    def _():
        pltpu.sync_copy(acc_sp.at[pl.ds(s * RPT, RPT)],
                        acc_hbm.at[pl.ds(c * N + s * RPT, RPT)])

    @pl.when(s == NS - 1)
    def _():
        pltpu.sync_copy(acc_sp.at[pl.ds(15 * RPT, RPT_LAST)],
                        acc_hbm.at[pl.ds(c * N + 15 * RPT, RPT_LAST)])


@functools.partial(
    pl.kernel,
    out_type=jax.ShapeDtypeStruct((2 * N, HALF), jnp.float32),
    mesh=_mesh,
    scratch_types=[
        pltpu.VMEM((NPH * CPP, CH), jnp.int32),
        pltpu.VMEM((CH, HALF), jnp.float32),
        pltpu.VMEM_SHARED((N + 8, HALF), jnp.float32),
    ],
)
def _deg_kernel(dst_hbm, ones_hbm, zeros_hbm, deg_hbm, dst_loc, ones_v, acc_sp):
    c = lax.axis_index("c")
    s = lax.axis_index("s")

    _zero_acc(zeros_hbm, acc_sp, s)
    pltpu.sync_copy(dst_hbm.at[s], dst_loc)
    pltpu.sync_copy(ones_hbm, ones_v)
    plsc.subcore_barrier()

    half = NPH * CPP // 2

    def body(g, carry):
        pltpu.sync_copy(ones_v, acc_sp.at[dst_loc.at[c * half + g]], add=True)
        return carry

    lax.fori_loop(0, half, body, 0)
    plsc.subcore_barrier()
    _writeback(acc_sp, deg_hbm, c, s)


@functools.partial(
    pl.kernel,
    out_type=jax.ShapeDtypeStruct((2 * N, HALF), jnp.float32),
    mesh=_mesh,
    scratch_types=[
        pltpu.VMEM((CPP, CH), jnp.int32),
        pltpu.VMEM((CPP, CH), jnp.int32),
        pltpu.VMEM((2, CH, HALF), jnp.float32),
        pltpu.SemaphoreType.DMA((2,)),
        pltpu.VMEM_SHARED((N + 8, HALF), jnp.float32),
    ],
)
def _agg_kernel(hs_hbm, src_hbm, dst_hbm, zeros_hbm, acc_hbm,
                src_loc, dst_loc, rowbuf, sem, acc_sp):
    c = lax.axis_index("c")
    s = lax.axis_index("s")

    _zero_acc(zeros_hbm, acc_sp, s)
    plsc.subcore_barrier()

    for p in range(NPH):
        pltpu.sync_copy(src_hbm.at[c, s, p], src_loc)
        pltpu.sync_copy(dst_hbm.at[s, p], dst_loc)

        def prime(g, carry):
            pltpu.async_copy(hs_hbm.at[src_loc.at[g]], rowbuf.at[g], sem.at[g])
            return carry

        lax.fori_loop(0, 2, prime, 0)

        def body(g, carry):
            b = lax.rem(g, 2)
            pltpu.make_async_copy(zeros_hbm.at[pl.ds(0, CH)],
                                  rowbuf.at[b], sem.at[b]).wait()
            pltpu.sync_copy(rowbuf.at[b], acc_sp.at[dst_loc.at[g]], add=True)
            pltpu.async_copy(hs_hbm.at[src_loc.at[g + 2]],
                             rowbuf.at[b], sem.at[b])
            return carry

        lax.fori_loop(0, CPP - 2, body, 0)

        def tail(g, carry):
            b = lax.rem(g, 2)
            pltpu.make_async_copy(zeros_hbm.at[pl.ds(0, CH)],
                                  rowbuf.at[b], sem.at[b]).wait()
            pltpu.sync_copy(rowbuf.at[b], acc_sp.at[dst_loc.at[g]], add=True)
            return carry

        lax.fori_loop(CPP - 2, CPP, tail, 0)

    plsc.subcore_barrier()
    _writeback(acc_sp, acc_hbm, c, s)


def _mm1_body(x_ref, w_ref, deg_ref, dhi_ref, h_ref, hs_ref):
    h = jnp.dot(x_ref[...], w_ref[...], preferred_element_type=jnp.float32)
    dinv = lax.rsqrt(deg_ref[:, 0:1] + dhi_ref[:, 0:1] + 1.0)
    h_ref[...] = h
    hs_ref[...] = h * dinv


_mm1 = pl.pallas_call(
    _mm1_body,
    grid=(NBLK, 2),
    in_specs=[
        pl.BlockSpec((BN, D), lambda i, j: (i, 0)),
        pl.BlockSpec((D, HALF), lambda i, j: (0, j)),
        pl.BlockSpec((BN, HALF), lambda i, j: (i, 0)),
        pl.BlockSpec((BN, HALF), lambda i, j: (NBLK + i, 0)),
    ],
    out_specs=[
        pl.BlockSpec((BN, HALF), lambda i, j: (j * NBLK + i, 0)),
        pl.BlockSpec((BN, HALF), lambda i, j: (j * NBLK + i, 0)),
    ],
    out_shape=[
        jax.ShapeDtypeStruct((2 * N, HALF), jnp.float32),
        jax.ShapeDtypeStruct((2 * N, HALF), jnp.float32),
    ],
)


def _mm_mid_body(alo_ref, ahi_ref, hlo_ref, hhi_ref, deg_ref, dhi_ref,
                 b_ref, w_ref, h_ref, hs_ref):
    dinv = lax.rsqrt(deg_ref[:, 0:1] + dhi_ref[:, 0:1] + 1.0)
    acc = jnp.concatenate([alo_ref[...], ahi_ref[...]], axis=1)
    hp = jnp.concatenate([hlo_ref[...], hhi_ref[...]], axis=1)
    z = jnp.maximum(dinv * acc + (dinv * dinv) * hp + b_ref[...], 0.0)
    h = jnp.dot(z, w_ref[...], preferred_element_type=jnp.float32)
    h_ref[...] = h
    hs_ref[...] = h * dinv


_mm_mid = pl.pallas_call(
    _mm_mid_body,
    grid=(NBLK, 2),
    in_specs=[
        pl.BlockSpec((BN, HALF), lambda i, j: (i, 0)),
        pl.BlockSpec((BN, HALF), lambda i, j: (NBLK + i, 0)),
        pl.BlockSpec((BN, HALF), lambda i, j: (i, 0)),
        pl.BlockSpec((BN, HALF), lambda i, j: (NBLK + i, 0)),
        pl.BlockSpec((BN, HALF), lambda i, j: (i, 0)),
        pl.BlockSpec((BN, HALF), lambda i, j: (NBLK + i, 0)),
        pl.BlockSpec((1, D), lambda i, j: (0, 0)),
        pl.BlockSpec((D, HALF), lambda i, j: (0, j)),
    ],
    out_specs=[
        pl.BlockSpec((BN, HALF), lambda i, j: (j * NBLK + i, 0)),
        pl.BlockSpec((BN, HALF), lambda i, j: (j * NBLK + i, 0)),
    ],
    out_shape=[
        jax.ShapeDtypeStruct((2 * N, HALF), jnp.float32),
        jax.ShapeDtypeStruct((2 * N, HALF), jnp.float32),
    ],
)


def _final_body(alo_ref, ahi_ref, hlo_ref, hhi_ref, deg_ref, dhi_ref,
                b_ref, out_ref):
    dinv = lax.rsqrt(deg_ref[:, 0:1] + dhi_ref[:, 0:1] + 1.0)
    acc = jnp.concatenate([alo_ref[...], ahi_ref[...]], axis=1)
    hp = jnp.concatenate([hlo_ref[...], hhi_ref[...]], axis=1)
    out_ref[...] = dinv * acc + (dinv * dinv) * hp + b_ref[...]


_final = pl.pallas_call(
    _final_body,
    grid=(NBLK,),
    in_specs=[
        pl.BlockSpec((BN, HALF), lambda i: (i, 0)),
        pl.BlockSpec((BN, HALF), lambda i: (NBLK + i, 0)),
        pl.BlockSpec((BN, HALF), lambda i: (i, 0)),
        pl.BlockSpec((BN, HALF), lambda i: (NBLK + i, 0)),
        pl.BlockSpec((BN, HALF), lambda i: (i, 0)),
        pl.BlockSpec((BN, HALF), lambda i: (NBLK + i, 0)),
        pl.BlockSpec((1, D), lambda i: (0, 0)),
    ],
    out_specs=pl.BlockSpec((BN, D), lambda i: (i, 0)),
    out_shape=jax.ShapeDtypeStruct((N, D), jnp.float32),
)


def kernel(x, edge_index, W1, b1, W2, b2, W3, b3):
    srcp = jnp.pad(edge_index[0].reshape(NS, EPT), ((0, 0), (0, PADT)))
    dstp = jnp.pad(edge_index[1].reshape(NS, EPT), ((0, 0), (0, PADT)),
                   constant_values=N)
    src = srcp.reshape(NS, NPH, CPP, CH)
    dst = dstp.reshape(NS, NPH, CPP, CH)
    src_off = src[None] + (jnp.arange(NC, dtype=jnp.int32) * N)[:, None, None, None, None]

    zeros_r = jnp.zeros((RPT_LAST, HALF), jnp.float32)
    b1r = b1.reshape(1, D)
    b2r = b2.reshape(1, D)
    b3r = b3.reshape(1, D)

    ones_c = jnp.ones((CH, HALF), jnp.float32)
    deg = _deg_kernel(dst.reshape(NS, NPH * CPP, CH), ones_c, zeros_r)
    h1, hs1 = _mm1(x, W1, deg, deg)
    acc1 = _agg_kernel(hs1, src_off, dst, zeros_r)
    h2, hs2 = _mm_mid(acc1, acc1, h1, h1, deg, deg, b1r, W2)
    acc2 = _agg_kernel(hs2, src_off, dst, zeros_r)
    h3, hs3 = _mm_mid(acc2, acc2, h2, h2, deg, deg, b2r, W3)
    acc3 = _agg_kernel(hs3, src_off, dst, zeros_r)
    return _final(acc3, acc3, h3, h3, deg, deg, b3r)
